# Initial kernel scaffold; baseline (speedup 1.0000x reference)
import jax
import jax.numpy as jnp
from jax.experimental import pallas as pl


def kernel(input, table):
    raise NotImplementedError("write your pallas kernel here")


# speedup vs baseline: 4.0873x; 4.0873x over previous
import functools

import jax
import jax.numpy as jnp
from jax import lax
from jax.experimental import pallas as pl
from jax.experimental.pallas import tpu as pltpu
from jax.experimental.pallas import tpu_sc as plsc

_VOCAB = 100000
_EMBED_DIM = 64
_BATCH = 4096
_HIST = 50

_CHUNK = 128
_TOTAL = _BATCH * _HIST
_NCHUNKS = _TOTAL // _CHUNK
_NW = 32
_CPW = _NCHUNKS // _NW


def _body(idx_hbm, table_hbm, out_hbm, idx_v, rows_v, gsem):
    wid = lax.axis_index("s") * 2 + lax.axis_index("c")
    base = wid * _CPW
    pltpu.sync_copy(idx_hbm.at[wid], idx_v)

    @pl.loop(0, _CPW)
    def chunk(j):
        pltpu.async_copy(table_hbm.at[idx_v.at[j]], rows_v, gsem).wait()
        pltpu.sync_copy(rows_v, out_hbm.at[base + j])


@jax.jit
def _embed(idx3d, table):
    mesh = plsc.VectorSubcoreMesh(core_axis_name="c", subcore_axis_name="s")
    f = pl.kernel(
        _body,
        out_type=jax.ShapeDtypeStruct((_NCHUNKS, _CHUNK, _EMBED_DIM), jnp.float32),
        mesh=mesh,
        scratch_types=[
            pltpu.VMEM((_CPW, _CHUNK), jnp.int32),
            pltpu.VMEM((_CHUNK, _EMBED_DIM), jnp.float32),
            pltpu.SemaphoreType.DMA,
        ],
        compiler_params=pltpu.CompilerParams(use_tc_tiling_on_sc=False),
    )
    return f(idx3d, table)


def kernel(input, table):
    idx3d = input.reshape(_NW, _CPW, _CHUNK).astype(jnp.int32)
    out = _embed(idx3d, table)
    return out.reshape(_BATCH, _HIST, _EMBED_DIM)


# speedup vs baseline: 4.6570x; 1.1394x over previous
import jax
import jax.numpy as jnp
from jax import lax
from jax.experimental import pallas as pl
from jax.experimental.pallas import tpu as pltpu
from jax.experimental.pallas import tpu_sc as plsc

_VOCAB = 100000
_EMBED_DIM = 64
_BATCH = 4096
_HIST = 50

_CHUNK = 128
_TOTAL = _BATCH * _HIST
_NCHUNKS = _TOTAL // _CHUNK
_NW = 32
_CPW = _NCHUNKS // _NW
_NBUF = 10
_NGRP = _CPW // _NBUF


def _body(idx_hbm, table_hbm, out_hbm, idx_v, rows_v, *sems):
    gsem = sems[:_NBUF]
    wsem = sems[_NBUF:]
    wid = lax.axis_index("s") * 2 + lax.axis_index("c")
    base = wid * _CPW
    pltpu.sync_copy(idx_hbm.at[wid], idx_v)

    def gather(chunk_j, b):
        return pltpu.make_async_copy(
            table_hbm.at[idx_v.at[chunk_j]], rows_v.at[b], gsem[b])

    def write(chunk_j, b):
        return pltpu.make_async_copy(
            rows_v.at[b], out_hbm.at[base + chunk_j], wsem[b])

    for b in range(_NBUF):
        gather(b, b).start()

    @pl.loop(0, _NGRP - 1)
    def grp(g):
        j0 = g * _NBUF
        for b in range(_NBUF):
            gather(j0 + b, b).wait()
            write(j0 + b, b).start()
        for b in range(_NBUF):
            write(j0 + b, b).wait()
            gather(j0 + _NBUF + b, b).start()

    j0 = (_NGRP - 1) * _NBUF
    for b in range(_NBUF):
        gather(j0 + b, b).wait()
        write(j0 + b, b).start()
    for b in range(_NBUF):
        write(j0 + b, b).wait()


@jax.jit
def _embed(idx3d, table):
    mesh = plsc.VectorSubcoreMesh(core_axis_name="c", subcore_axis_name="s")
    f = pl.kernel(
        _body,
        out_type=jax.ShapeDtypeStruct((_NCHUNKS, _CHUNK, _EMBED_DIM), jnp.float32),
        mesh=mesh,
        scratch_types=[
            pltpu.VMEM((_CPW, _CHUNK), jnp.int32),
            pltpu.VMEM((_NBUF, _CHUNK, _EMBED_DIM), jnp.float32),
        ] + [pltpu.SemaphoreType.DMA] * (2 * _NBUF),
        compiler_params=pltpu.CompilerParams(use_tc_tiling_on_sc=False),
    )
    return f(idx3d, table)


def kernel(input, table):
    idx3d = input.reshape(_NW, _CPW, _CHUNK).astype(jnp.int32)
    out = _embed(idx3d, table)
    return out.reshape(_BATCH, _HIST, _EMBED_DIM)


# speedup vs baseline: 4.6682x; 1.0024x over previous
import jax
import jax.numpy as jnp
from jax import lax
from jax.experimental import pallas as pl
from jax.experimental.pallas import tpu as pltpu
from jax.experimental.pallas import tpu_sc as plsc

_VOCAB = 100000
_EMBED_DIM = 64
_BATCH = 4096
_HIST = 50

_CHUNK = 128
_TOTAL = _BATCH * _HIST
_NCHUNKS = _TOTAL // _CHUNK
_NW = 32
_CPW = _NCHUNKS // _NW
_NBUF = 10
_NGRP = _CPW // _NBUF


def _body(idx_hbm, table_hbm, out_hbm, idx_v, rows_v, *sems):
    gsem = sems[:_NBUF]
    wsem = sems[_NBUF:]
    wid = lax.axis_index("s") * 2 + lax.axis_index("c")
    base = wid * _CPW
    pltpu.sync_copy(idx_hbm.at[pl.ds(base, _CPW)], idx_v)

    def gather(chunk_j, b):
        return pltpu.make_async_copy(
            table_hbm.at[idx_v.at[chunk_j]], rows_v.at[b], gsem[b])

    def write(chunk_j, b):
        return pltpu.make_async_copy(
            rows_v.at[b], out_hbm.at[base + chunk_j], wsem[b])

    for b in range(_NBUF):
        gather(b, b).start()

    @pl.loop(0, _NGRP - 1)
    def grp(g):
        j0 = g * _NBUF
        for b in range(_NBUF):
            gather(j0 + b, b).wait()
            write(j0 + b, b).start()
        for b in range(_NBUF):
            write(j0 + b, b).wait()
            gather(j0 + _NBUF + b, b).start()

    j0 = (_NGRP - 1) * _NBUF
    for b in range(_NBUF):
        gather(j0 + b, b).wait()
        write(j0 + b, b).start()
    for b in range(_NBUF):
        write(j0 + b, b).wait()


@jax.jit
def _embed(idx2, table):
    mesh = plsc.VectorSubcoreMesh(core_axis_name="c", subcore_axis_name="s")
    f = pl.kernel(
        _body,
        out_type=jax.ShapeDtypeStruct((_NCHUNKS, _CHUNK, _EMBED_DIM), jnp.float32),
        mesh=mesh,
        scratch_types=[
            pltpu.VMEM((_CPW, _CHUNK), jnp.int32),
            pltpu.VMEM((_NBUF, _CHUNK, _EMBED_DIM), jnp.float32),
        ] + [pltpu.SemaphoreType.DMA] * (2 * _NBUF),
        compiler_params=pltpu.CompilerParams(use_tc_tiling_on_sc=False),
    )
    return f(idx2, table)


def kernel(input, table):
    idx2 = input.reshape(_NCHUNKS, _CHUNK).astype(jnp.int32)
    out = _embed(idx2, table)
    return out.reshape(_BATCH, _HIST, _EMBED_DIM)
